# Initial kernel scaffold; baseline (speedup 1.0000x reference)
import jax
import jax.numpy as jnp
from jax.experimental import pallas as pl


def kernel(x, edge_index, batch, W1, b1, W2, b2, LW1, Lb1, LW2, Lb2):
    raise NotImplementedError("write your pallas kernel here")


# speedup vs baseline: 45.4476x; 45.4476x over previous
import functools

import jax
import jax.numpy as jnp
from jax import lax
from jax.experimental import pallas as pl
from jax.experimental.pallas import tpu as pltpu
from jax.experimental.pallas import tpu_sc as plsc

N = 100000
E = 3200000
G = 128
CFG = 64

NC = 2
NS = 16
LANE = 128

N_PAD = 100352
NP8 = N_PAD // 8
E_ROWS = 25344
E_PAD = E_ROWS * LANE

AGG_CH = 6
AGG_ROWS_PT = E_ROWS // NS
AGG_NCH = AGG_ROWS_PT // AGG_CH

DEG_CH = 12
DEG_ROWS_PT = E_ROWS // (NC * NS)
DEG_NCH = DEG_ROWS_PT // DEG_CH

ACC_ROWS_PT = N_PAD // NS
ZCH = 392

RCH = 784
GP = 132

TCB = 256
NEG = -3.4e38

_mesh = plsc.VectorSubcoreMesh(
    core_axis_name="c", subcore_axis_name="s", num_cores=NC, num_subcores=NS)

_sc_params = pltpu.CompilerParams(use_tc_tiling_on_sc=False)


def _zero_f32(ref, nrows):
    def body(r, _):
        ref[r] = jnp.zeros((16,), jnp.float32)
        return 0
    lax.fori_loop(0, nrows, body, 0)


@functools.partial(
    pl.kernel,
    out_type=jax.ShapeDtypeStruct((NC, N_PAD), jnp.float32),
    mesh=_mesh,
    compiler_params=_sc_params,
    scratch_types=[
        pltpu.VMEM_SHARED((N_PAD,), jnp.float32),
        pltpu.VMEM((DEG_CH, LANE), jnp.int32),
        pltpu.VMEM((DEG_CH, LANE), jnp.int32),
        pltpu.VMEM((DEG_CH, LANE), jnp.float32),
        pltpu.VMEM((ACC_ROWS_PT,), jnp.float32),
        pltpu.SemaphoreType.DMA,
        pltpu.SemaphoreType.DMA,
    ],
)
def _deg_kernel(dst_m, out, acc, di0, di1, ones, zb, sem0, sem1):
    cid = lax.axis_index("c")
    sid = lax.axis_index("s")
    w = cid * NS + sid

    for r in range(DEG_CH):
        for j in range(LANE // 16):
            ones[r, pl.ds(j * 16, 16)] = jnp.ones((16,), jnp.float32)

    def zbody(i, _):
        zb[pl.ds(i * 16, 16)] = jnp.zeros((16,), jnp.float32)
        return 0
    lax.fori_loop(0, ACC_ROWS_PT // 16, zbody, 0)
    pltpu.sync_copy(zb, acc.at[pl.ds(sid * ACC_ROWS_PT, ACC_ROWS_PT)])
    plsc.subcore_barrier()

    def stage(c, di):
        base = w * DEG_ROWS_PT + c * DEG_CH
        pltpu.sync_copy(dst_m.at[pl.ds(base, DEG_CH)], di)

    def fire(di, sem):
        for r in range(DEG_CH):
            pltpu.async_copy(ones.at[r], acc.at[di.at[r]], sem, add=True)

    def drain(di, sem):
        for r in range(DEG_CH):
            pltpu.make_async_copy(ones.at[r], acc.at[di.at[r]], sem).wait()

    def pair(k, _):
        c = 2 * k

        @pl.when(k > 0)
        def _():
            drain(di0, sem0)
        stage(c, di0)
        fire(di0, sem0)

        @pl.when(k > 0)
        def _():
            drain(di1, sem1)
        stage(c + 1, di1)
        fire(di1, sem1)
        return 0
    lax.fori_loop(0, DEG_NCH // 2, pair, 0)
    drain(di0, sem0)
    drain(di1, sem1)

    plsc.subcore_barrier()
    pltpu.sync_copy(acc.at[pl.ds(sid * ACC_ROWS_PT, ACC_ROWS_PT)], zb)
    pltpu.sync_copy(zb, out.at[cid, pl.ds(sid * ACC_ROWS_PT, ACC_ROWS_PT)])


@functools.partial(
    pl.kernel,
    out_type=(jax.ShapeDtypeStruct((N_PAD, 16), jnp.float32),
              jax.ShapeDtypeStruct((N_PAD, 16), jnp.float32)),
    mesh=_mesh,
    compiler_params=_sc_params,
    scratch_types=[
        pltpu.VMEM_SHARED((N_PAD, 16), jnp.float32),
        pltpu.VMEM((AGG_CH, LANE), jnp.int32),
        pltpu.VMEM((AGG_CH, LANE), jnp.int32),
        pltpu.VMEM((AGG_CH, LANE), jnp.int32),
        pltpu.VMEM((AGG_CH, LANE), jnp.int32),
        pltpu.VMEM((AGG_CH * LANE, 16), jnp.float32),
        pltpu.VMEM((AGG_CH * LANE, 16), jnp.float32),
        pltpu.SemaphoreType.DMA,
        pltpu.SemaphoreType.DMA,
        pltpu.SemaphoreType.DMA,
        pltpu.SemaphoreType.DMA,
    ],
)
def _agg_kernel(y_a, y_b, src_m, dst_m, out_a, out_b,
                acc, si0, di0, si1, di1, rows0, rows1,
                gs0, gs1, ss0, ss1):
    cid = lax.axis_index("c")
    sid = lax.axis_index("s")

    _zero_f32(rows0, ZCH)
    for k in range(ACC_ROWS_PT // ZCH):
        pltpu.sync_copy(
            rows0.at[pl.ds(0, ZCH)],
            acc.at[pl.ds(sid * ACC_ROWS_PT + k * ZCH, ZCH)])
    plsc.subcore_barrier()

    def gather_scatter(y_ref):
        bufs = ((si0, di0, rows0, gs0, ss0), (si1, di1, rows1, gs1, ss1))

        def stage(c, si, di):
            base = sid * AGG_ROWS_PT + c * AGG_CH
            pltpu.sync_copy(src_m.at[pl.ds(base, AGG_CH)], si)
            pltpu.sync_copy(dst_m.at[pl.ds(base, AGG_CH)], di)

        def fire_g(si, ro, gs):
            for r in range(AGG_CH):
                pltpu.async_copy(y_ref.at[si.at[r]],
                                 ro.at[pl.ds(r * LANE, LANE)], gs)

        def wait_g(si, ro, gs):
            for r in range(AGG_CH):
                pltpu.make_async_copy(y_ref.at[si.at[r]],
                                      ro.at[pl.ds(r * LANE, LANE)], gs).wait()

        def fire_s(di, ro, ss):
            for r in range(AGG_CH):
                pltpu.async_copy(ro.at[pl.ds(r * LANE, LANE)],
                                 acc.at[di.at[r]], ss, add=True)

        def wait_s(di, ro, ss):
            for r in range(AGG_CH):
                pltpu.make_async_copy(ro.at[pl.ds(r * LANE, LANE)],
                                      acc.at[di.at[r]], ss).wait()

        stage(0, si0, di0)
        fire_g(si0, rows0, gs0)

        def half(c, p):
            sip, dip, rop, gsp, ssp = bufs[p]
            siq, diq, roq, gsq, ssq = bufs[1 - p]
            wait_g(sip, rop, gsp)
            fire_s(dip, rop, ssp)

            @pl.when(c > 0)
            def _():
                wait_s(diq, roq, ssq)

            @pl.when(c + 1 < AGG_NCH)
            def _():
                stage(c + 1, siq, diq)
                fire_g(siq, roq, gsq)

        def pair(k, _):
            half(2 * k, 0)
            half(2 * k + 1, 1)
            return 0
        lax.fori_loop(0, AGG_NCH // 2, pair, 0)
        wait_s(di1, rows1, ss1)

    @pl.when(cid == 0)
    def _():
        gather_scatter(y_a)

    @pl.when(cid == 1)
    def _():
        gather_scatter(y_b)

    plsc.subcore_barrier()

    def writeback(out_ref):
        for k in range(ACC_ROWS_PT // ZCH):
            row0 = sid * ACC_ROWS_PT + k * ZCH
            pltpu.sync_copy(acc.at[pl.ds(row0, ZCH)], rows0.at[pl.ds(0, ZCH)])
            pltpu.sync_copy(rows0.at[pl.ds(0, ZCH)],
                            out_ref.at[pl.ds(row0, ZCH)])

    @pl.when(cid == 0)
    def _():
        writeback(out_a)

    @pl.when(cid == 1)
    def _():
        writeback(out_b)


_POOL_OUT_T = (
    jax.ShapeDtypeStruct((NC, NS, G, 16), jnp.float32),
    jax.ShapeDtypeStruct((NC, NS, G, 16), jnp.float32),
    jax.ShapeDtypeStruct((NC, NS, G, 16), jnp.float32),
)


@functools.partial(
    pl.kernel,
    out_type=_POOL_OUT_T,
    mesh=_mesh,
    compiler_params=_sc_params,
    scratch_types=[
        pltpu.VMEM((RCH, 16), jnp.float32),
        pltpu.VMEM((RCH,), jnp.int32),
        pltpu.VMEM((GP, 16), jnp.float32),
        pltpu.VMEM((GP, 16), jnp.float32),
        pltpu.VMEM((GP, 16), jnp.float32),
    ],
)
def _pool_kernel(h_a, h_b, batch_p, mx_o, sm_o, ct_o,
                 hbuf, bbuf, mxl, sml, ctl):
    cid = lax.axis_index("c")
    sid = lax.axis_index("s")

    def init(r, _):
        mxl[r] = jnp.full((16,), NEG, jnp.float32)
        sml[r] = jnp.zeros((16,), jnp.float32)
        ctl[r] = jnp.zeros((16,), jnp.float32)
        return 0
    lax.fori_loop(0, GP, init, 0)

    def sweep(h_ref):
        def chunk_body(k, _):
            row0 = sid * ACC_ROWS_PT + k * RCH
            pltpu.sync_copy(h_ref.at[pl.ds(row0, RCH)], hbuf)
            pltpu.sync_copy(batch_p.at[pl.ds(row0, RCH)], bbuf)

            def group_body(j, _):
                base = j * 16
                gv = bbuf[pl.ds(base, 16)]
                g0 = gv[0]
                g15 = gv[15]

                @pl.when(g0 == g15)
                def _():
                    v = [hbuf[base + i] for i in range(16)]
                    m = v
                    while len(m) > 1:
                        m = [jnp.maximum(m[2 * t], m[2 * t + 1])
                             for t in range(len(m) // 2)]
                    s = v
                    while len(s) > 1:
                        s = [s[2 * t] + s[2 * t + 1]
                             for t in range(len(s) // 2)]
                    mxl[g0] = jnp.maximum(mxl[g0], m[0])
                    sml[g0] = sml[g0] + s[0]
                    ctl[g0] = ctl[g0] + jnp.full((16,), 16.0, jnp.float32)

                @pl.when(g0 != g15)
                def _():
                    for i in range(16):
                        g = gv[i]
                        h = hbuf[base + i]
                        mxl[g] = jnp.maximum(mxl[g], h)
                        sml[g] = sml[g] + h
                        ctl[g] = ctl[g] + jnp.ones((16,), jnp.float32)
                return 0
            lax.fori_loop(0, RCH // 16, group_body, 0)
            return 0
        lax.fori_loop(0, ACC_ROWS_PT // RCH, chunk_body, 0)

    @pl.when(cid == 0)
    def _():
        sweep(h_a)

    @pl.when(cid == 1)
    def _():
        sweep(h_b)

    pltpu.sync_copy(mxl.at[pl.ds(0, G)], mx_o.at[cid, sid])
    pltpu.sync_copy(sml.at[pl.ds(0, G)], sm_o.at[cid, sid])
    pltpu.sync_copy(ctl.at[pl.ds(0, G)], ct_o.at[cid, sid])


def _blk(i):
    return (i, 0)


def _const(i):
    return (0, 0)


_ROW_SPEC = pl.BlockSpec((TCB, LANE), _blk)
_MAT_SPEC = pl.BlockSpec((LANE, LANE), _const)
_ROWV_SHAPE = jax.ShapeDtypeStruct((NP8, LANE), jnp.float32)


def _prep_body(cnt_ref, x_ref, bda_ref, bdb_ref, dinv_ref, ya_ref, yb_ref):
    deg = cnt_ref[0] + cnt_ref[1] + 1.0
    dinv = lax.rsqrt(deg)
    dinv_ref[...] = dinv
    ya_ref[...] = jnp.dot(x_ref[...], bda_ref[...],
                          preferred_element_type=jnp.float32) * dinv
    yb_ref[...] = jnp.dot(x_ref[...], bdb_ref[...],
                          preferred_element_type=jnp.float32) * dinv


_prep_tc = pl.pallas_call(
    _prep_body,
    grid=(NP8 // TCB,),
    in_specs=[
        pl.BlockSpec((NC, TCB, LANE), lambda i: (0, i, 0)),
        _ROW_SPEC, _MAT_SPEC, _MAT_SPEC,
    ],
    out_specs=[_ROW_SPEC, _ROW_SPEC, _ROW_SPEC],
    out_shape=[_ROWV_SHAPE, _ROWV_SHAPE, _ROWV_SHAPE],
)


def _mid1_body(aa_ref, ab_ref, ya_ref, yb_ref, dinv_ref, ba_ref, bb_ref,
               w_aa, w_ab, w_ba, w_bb,
               ha_ref, hb_ref, y2a_ref, y2b_ref):
    dinv = dinv_ref[...]
    ha = jnp.maximum((aa_ref[...] + ya_ref[...]) * dinv + ba_ref[...], 0.0)
    hb = jnp.maximum((ab_ref[...] + yb_ref[...]) * dinv + bb_ref[...], 0.0)
    ha_ref[...] = ha
    hb_ref[...] = hb
    y2a_ref[...] = (jnp.dot(ha, w_aa[...], preferred_element_type=jnp.float32)
                    + jnp.dot(hb, w_ba[...],
                              preferred_element_type=jnp.float32)) * dinv
    y2b_ref[...] = (jnp.dot(ha, w_ab[...], preferred_element_type=jnp.float32)
                    + jnp.dot(hb, w_bb[...],
                              preferred_element_type=jnp.float32)) * dinv


_mid1_tc = pl.pallas_call(
    _mid1_body,
    grid=(NP8 // TCB,),
    in_specs=[
        _ROW_SPEC, _ROW_SPEC, _ROW_SPEC, _ROW_SPEC, _ROW_SPEC,
        pl.BlockSpec((1, LANE), _const), pl.BlockSpec((1, LANE), _const),
        _MAT_SPEC, _MAT_SPEC, _MAT_SPEC, _MAT_SPEC,
    ],
    out_specs=[_ROW_SPEC, _ROW_SPEC, _ROW_SPEC, _ROW_SPEC],
    out_shape=[_ROWV_SHAPE, _ROWV_SHAPE, _ROWV_SHAPE, _ROWV_SHAPE],
)


def _mid2_body(aa_ref, ab_ref, ya_ref, yb_ref, dinv_ref, ba_ref, bb_ref,
               ha_ref, hb_ref):
    dinv = dinv_ref[...]
    ha_ref[...] = jnp.maximum(
        (aa_ref[...] + ya_ref[...]) * dinv + ba_ref[...], 0.0)
    hb_ref[...] = jnp.maximum(
        (ab_ref[...] + yb_ref[...]) * dinv + bb_ref[...], 0.0)


_mid2_tc = pl.pallas_call(
    _mid2_body,
    grid=(NP8 // TCB,),
    in_specs=[
        _ROW_SPEC, _ROW_SPEC, _ROW_SPEC, _ROW_SPEC, _ROW_SPEC,
        pl.BlockSpec((1, LANE), _const), pl.BlockSpec((1, LANE), _const),
    ],
    out_specs=[_ROW_SPEC, _ROW_SPEC],
    out_shape=[_ROWV_SHAPE, _ROWV_SHAPE],
)


def _head_body(mx1_ref, sm1_ref, mx2_ref, sm2_ref, ct_ref,
               lw1_ref, lb1_ref, lw2_ref, lb2_ref, out_ref):
    mx1 = mx1_ref[...]
    sm1 = sm1_ref[...]
    mx2 = mx2_ref[...]
    sm2 = sm2_ref[...]
    cnt = jnp.maximum(jnp.sum(ct_ref[...], axis=0), 1.0)
    parts = [
        jnp.max(mx1[0], axis=0) + jnp.max(mx2[0], axis=0),
        jnp.max(mx1[1], axis=0) + jnp.max(mx2[1], axis=0),
        (jnp.sum(sm1[0], axis=0) + jnp.sum(sm2[0], axis=0)) / cnt,
        (jnp.sum(sm1[1], axis=0) + jnp.sum(sm2[1], axis=0)) / cnt,
    ]
    lw1 = lw1_ref[...]
    z = lb1_ref[...]
    for p, xp in enumerate(parts):
        z = z + jnp.dot(xp, lw1[16 * p:16 * p + 16, :],
                        preferred_element_type=jnp.float32)
    z = jnp.maximum(z, 0.0)
    out_ref[...] = jnp.dot(z, lw2_ref[...],
                           preferred_element_type=jnp.float32) + lb2_ref[...]


_PART_SPEC = pl.BlockSpec((NC, NS, G, 16), lambda: (0, 0, 0, 0))

_head_tc = pl.pallas_call(
    _head_body,
    in_specs=[
        _PART_SPEC, _PART_SPEC, _PART_SPEC, _PART_SPEC,
        pl.BlockSpec((NS, G, 16), lambda: (0, 0, 0)),
        pl.BlockSpec((64, 64), lambda: (0, 0)),
        pl.BlockSpec((1, 64), lambda: (0, 0)),
        pl.BlockSpec((64, 64), lambda: (0, 0)),
        pl.BlockSpec((1, 64), lambda: (0, 0)),
    ],
    out_specs=pl.BlockSpec((G, CFG), lambda: (0, 0)),
    out_shape=jax.ShapeDtypeStruct((G, CFG), jnp.float32),
)


def _bd(w16):
    return jnp.kron(jnp.eye(8, dtype=jnp.float32), w16)


def kernel(x, edge_index, batch, W1, b1, W2, b2, LW1, Lb1, LW2, Lb2):
    src = edge_index[0]
    dst = edge_index[1]
    pad = N + (jnp.arange(E_PAD - E, dtype=jnp.int32) % 256)
    src_m = jnp.concatenate([src, pad]).reshape(E_ROWS, LANE)
    dst_m = jnp.concatenate([dst, pad]).reshape(E_ROWS, LANE)
    batch_p = jnp.concatenate(
        [batch, jnp.full((N_PAD - N,), G, jnp.int32)])

    x16 = jnp.zeros((N_PAD, 16), jnp.float32).at[:N, :3].set(x)
    x_pk = x16.reshape(NP8, LANE)

    W1p = jnp.zeros((16, 32), jnp.float32).at[:3].set(W1)
    bd1a = _bd(W1p[:, :16])
    bd1b = _bd(W1p[:, 16:])
    bd2aa = _bd(W2[:16, :16])
    bd2ab = _bd(W2[:16, 16:])
    bd2ba = _bd(W2[16:, :16])
    bd2bb = _bd(W2[16:, 16:])
    b1a = jnp.tile(b1[:16], 8).reshape(1, LANE)
    b1b = jnp.tile(b1[16:], 8).reshape(1, LANE)
    b2a = jnp.tile(b2[:16], 8).reshape(1, LANE)
    b2b = jnp.tile(b2[16:], 8).reshape(1, LANE)

    cnt = _deg_kernel(dst_m)
    cnt_pk = jnp.broadcast_to(cnt[:, :, None], (NC, N_PAD, 16)).reshape(
        NC, NP8, LANE)

    dinv_pk, y1a_pk, y1b_pk = _prep_tc(cnt_pk, x_pk, bd1a, bd1b)

    a1a, a1b = _agg_kernel(
        y1a_pk.reshape(N_PAD, 16), y1b_pk.reshape(N_PAD, 16), src_m, dst_m)

    h1a_pk, h1b_pk, y2a_pk, y2b_pk = _mid1_tc(
        a1a.reshape(NP8, LANE), a1b.reshape(NP8, LANE), y1a_pk, y1b_pk,
        dinv_pk, b1a, b1b, bd2aa, bd2ab, bd2ba, bd2bb)

    mx1, sm1, ct1 = _pool_kernel(
        h1a_pk.reshape(N_PAD, 16), h1b_pk.reshape(N_PAD, 16), batch_p)

    a2a, a2b = _agg_kernel(
        y2a_pk.reshape(N_PAD, 16), y2b_pk.reshape(N_PAD, 16), src_m, dst_m)

    h2a_pk, h2b_pk = _mid2_tc(
        a2a.reshape(NP8, LANE), a2b.reshape(NP8, LANE), y2a_pk, y2b_pk,
        dinv_pk, b2a, b2b)

    mx2, sm2, ct2 = _pool_kernel(
        h2a_pk.reshape(N_PAD, 16), h2b_pk.reshape(N_PAD, 16), batch_p)

    return _head_tc(mx1, sm1, mx2, sm2, ct1[0],
                    LW1, Lb1.reshape(1, 64), LW2, Lb2.reshape(1, 64))


# speedup vs baseline: 45.7223x; 1.0060x over previous
import functools

import jax
import jax.numpy as jnp
from jax import lax
from jax.experimental import pallas as pl
from jax.experimental.pallas import tpu as pltpu
from jax.experimental.pallas import tpu_sc as plsc

N = 100000
E = 3200000
G = 128
CFG = 64

NC = 2
NS = 16
LANE = 128

N_PAD = 100352
NP8 = N_PAD // 8
E_ROWS = 25344
E_PAD = E_ROWS * LANE

AGG_CH = 6
AGG_ROWS_PT = E_ROWS // NS
AGG_NCH = AGG_ROWS_PT // AGG_CH

DEG_CH = 12
DEG_ROWS_PT = E_ROWS // (NC * NS)
DEG_NCH = DEG_ROWS_PT // DEG_CH

ACC_ROWS_PT = N_PAD // NS
ZCH = 392

RCH = 784
GP = 132

TCB = 256
NEG = -3.4e38

_mesh = plsc.VectorSubcoreMesh(
    core_axis_name="c", subcore_axis_name="s", num_cores=NC, num_subcores=NS)

_sc_params = pltpu.CompilerParams(use_tc_tiling_on_sc=False)


def _zero_f32(ref, nrows):
    def body(r, _):
        ref[r] = jnp.zeros((16,), jnp.float32)
        return 0
    lax.fori_loop(0, nrows, body, 0)


@functools.partial(
    pl.kernel,
    out_type=jax.ShapeDtypeStruct((NC, N_PAD), jnp.float32),
    mesh=_mesh,
    compiler_params=_sc_params,
    scratch_types=[
        pltpu.VMEM_SHARED((N_PAD,), jnp.float32),
        pltpu.VMEM((DEG_CH * LANE,), jnp.int32),
        pltpu.VMEM((DEG_CH * LANE,), jnp.int32),
        pltpu.VMEM((DEG_CH * LANE,), jnp.float32),
        pltpu.VMEM((ACC_ROWS_PT,), jnp.float32),
        pltpu.SemaphoreType.DMA,
        pltpu.SemaphoreType.DMA,
    ],
)
def _deg_kernel(dst_v, out, acc, di0, di1, ones, zb, sem0, sem1):
    cid = lax.axis_index("c")
    sid = lax.axis_index("s")
    w = cid * NS + sid

    def obody(i, _):
        ones[pl.ds(i * 16, 16)] = jnp.ones((16,), jnp.float32)
        return 0
    lax.fori_loop(0, DEG_CH * LANE // 16, obody, 0)

    def zbody(i, _):
        zb[pl.ds(i * 16, 16)] = jnp.zeros((16,), jnp.float32)
        return 0
    lax.fori_loop(0, ACC_ROWS_PT // 16, zbody, 0)
    pltpu.sync_copy(zb, acc.at[pl.ds(sid * ACC_ROWS_PT, ACC_ROWS_PT)])
    plsc.subcore_barrier()

    def stage(c, di):
        base = (w * DEG_ROWS_PT + c * DEG_CH) * LANE
        pltpu.sync_copy(dst_v.at[pl.ds(base, DEG_CH * LANE)], di)

    def fire(di, sem):
        pltpu.async_copy(ones, acc.at[di], sem, add=True)

    def drain(di, sem):
        pltpu.make_async_copy(ones, acc.at[di], sem).wait()

    def pair(k, _):
        c = 2 * k

        @pl.when(k > 0)
        def _():
            drain(di0, sem0)
        stage(c, di0)
        fire(di0, sem0)

        @pl.when(k > 0)
        def _():
            drain(di1, sem1)
        stage(c + 1, di1)
        fire(di1, sem1)
        return 0
    lax.fori_loop(0, DEG_NCH // 2, pair, 0)
    drain(di0, sem0)
    drain(di1, sem1)

    plsc.subcore_barrier()
    pltpu.sync_copy(acc.at[pl.ds(sid * ACC_ROWS_PT, ACC_ROWS_PT)], zb)
    pltpu.sync_copy(zb, out.at[cid, pl.ds(sid * ACC_ROWS_PT, ACC_ROWS_PT)])


@functools.partial(
    pl.kernel,
    out_type=(jax.ShapeDtypeStruct((N_PAD, 16), jnp.float32),
              jax.ShapeDtypeStruct((N_PAD, 16), jnp.float32)),
    mesh=_mesh,
    compiler_params=_sc_params,
    scratch_types=[
        pltpu.VMEM_SHARED((N_PAD, 16), jnp.float32),
        pltpu.VMEM((AGG_CH * LANE,), jnp.int32),
        pltpu.VMEM((AGG_CH * LANE,), jnp.int32),
        pltpu.VMEM((AGG_CH * LANE,), jnp.int32),
        pltpu.VMEM((AGG_CH * LANE,), jnp.int32),
        pltpu.VMEM((AGG_CH * LANE, 16), jnp.float32),
        pltpu.VMEM((AGG_CH * LANE, 16), jnp.float32),
        pltpu.SemaphoreType.DMA,
        pltpu.SemaphoreType.DMA,
        pltpu.SemaphoreType.DMA,
        pltpu.SemaphoreType.DMA,
    ],
)
def _agg_kernel(y_a, y_b, src_v, dst_v, out_a, out_b,
                acc, si0, di0, si1, di1, rows0, rows1,
                gs0, gs1, ss0, ss1):
    cid = lax.axis_index("c")
    sid = lax.axis_index("s")

    _zero_f32(rows0, ZCH)
    for k in range(ACC_ROWS_PT // ZCH):
        pltpu.sync_copy(
            rows0.at[pl.ds(0, ZCH)],
            acc.at[pl.ds(sid * ACC_ROWS_PT + k * ZCH, ZCH)])
    plsc.subcore_barrier()

    def gather_scatter(y_ref):
        bufs = ((si0, di0, rows0, gs0, ss0), (si1, di1, rows1, gs1, ss1))

        def stage(c, si, di):
            base = (sid * AGG_ROWS_PT + c * AGG_CH) * LANE
            pltpu.sync_copy(src_v.at[pl.ds(base, AGG_CH * LANE)], si)
            pltpu.sync_copy(dst_v.at[pl.ds(base, AGG_CH * LANE)], di)

        def fire_g(si, ro, gs):
            pltpu.async_copy(y_ref.at[si], ro, gs)

        def wait_g(si, ro, gs):
            pltpu.make_async_copy(y_ref.at[si], ro, gs).wait()

        def fire_s(di, ro, ss):
            pltpu.async_copy(ro, acc.at[di], ss, add=True)

        def wait_s(di, ro, ss):
            pltpu.make_async_copy(ro, acc.at[di], ss).wait()

        stage(0, si0, di0)
        fire_g(si0, rows0, gs0)

        def half(c, p):
            sip, dip, rop, gsp, ssp = bufs[p]
            siq, diq, roq, gsq, ssq = bufs[1 - p]
            wait_g(sip, rop, gsp)
            fire_s(dip, rop, ssp)

            @pl.when(c > 0)
            def _():
                wait_s(diq, roq, ssq)

            @pl.when(c + 1 < AGG_NCH)
            def _():
                stage(c + 1, siq, diq)
                fire_g(siq, roq, gsq)

        def pair(k, _):
            half(2 * k, 0)
            half(2 * k + 1, 1)
            return 0
        lax.fori_loop(0, AGG_NCH // 2, pair, 0)
        wait_s(di1, rows1, ss1)

    @pl.when(cid == 0)
    def _():
        gather_scatter(y_a)

    @pl.when(cid == 1)
    def _():
        gather_scatter(y_b)

    plsc.subcore_barrier()

    def writeback(out_ref):
        for k in range(ACC_ROWS_PT // ZCH):
            row0 = sid * ACC_ROWS_PT + k * ZCH
            pltpu.sync_copy(acc.at[pl.ds(row0, ZCH)], rows0.at[pl.ds(0, ZCH)])
            pltpu.sync_copy(rows0.at[pl.ds(0, ZCH)],
                            out_ref.at[pl.ds(row0, ZCH)])

    @pl.when(cid == 0)
    def _():
        writeback(out_a)

    @pl.when(cid == 1)
    def _():
        writeback(out_b)


_POOL_OUT_T = (
    jax.ShapeDtypeStruct((NC, NS, G, 16), jnp.float32),
    jax.ShapeDtypeStruct((NC, NS, G, 16), jnp.float32),
    jax.ShapeDtypeStruct((NC, NS, G, 16), jnp.float32),
)


@functools.partial(
    pl.kernel,
    out_type=_POOL_OUT_T,
    mesh=_mesh,
    compiler_params=_sc_params,
    scratch_types=[
        pltpu.VMEM((RCH, 16), jnp.float32),
        pltpu.VMEM((RCH,), jnp.int32),
        pltpu.VMEM((GP, 16), jnp.float32),
        pltpu.VMEM((GP, 16), jnp.float32),
        pltpu.VMEM((GP, 16), jnp.float32),
    ],
)
def _pool_kernel(h_a, h_b, batch_p, mx_o, sm_o, ct_o,
                 hbuf, bbuf, mxl, sml, ctl):
    cid = lax.axis_index("c")
    sid = lax.axis_index("s")

    def init(r, _):
        mxl[r] = jnp.full((16,), NEG, jnp.float32)
        sml[r] = jnp.zeros((16,), jnp.float32)
        ctl[r] = jnp.zeros((16,), jnp.float32)
        return 0
    lax.fori_loop(0, GP, init, 0)

    def sweep(h_ref):
        def chunk_body(k, _):
            row0 = sid * ACC_ROWS_PT + k * RCH
            pltpu.sync_copy(h_ref.at[pl.ds(row0, RCH)], hbuf)
            pltpu.sync_copy(batch_p.at[pl.ds(row0, RCH)], bbuf)

            def group_body(j, _):
                base = j * 16
                gv = bbuf[pl.ds(base, 16)]
                g0 = gv[0]
                g15 = gv[15]

                @pl.when(g0 == g15)
                def _():
                    v = [hbuf[base + i] for i in range(16)]
                    m = v
                    while len(m) > 1:
                        m = [jnp.maximum(m[2 * t], m[2 * t + 1])
                             for t in range(len(m) // 2)]
                    s = v
                    while len(s) > 1:
                        s = [s[2 * t] + s[2 * t + 1]
                             for t in range(len(s) // 2)]
                    mxl[g0] = jnp.maximum(mxl[g0], m[0])
                    sml[g0] = sml[g0] + s[0]
                    ctl[g0] = ctl[g0] + jnp.full((16,), 16.0, jnp.float32)

                @pl.when(g0 != g15)
                def _():
                    for i in range(16):
                        g = gv[i]
                        h = hbuf[base + i]
                        mxl[g] = jnp.maximum(mxl[g], h)
                        sml[g] = sml[g] + h
                        ctl[g] = ctl[g] + jnp.ones((16,), jnp.float32)
                return 0
            lax.fori_loop(0, RCH // 16, group_body, 0)
            return 0
        lax.fori_loop(0, ACC_ROWS_PT // RCH, chunk_body, 0)

    @pl.when(cid == 0)
    def _():
        sweep(h_a)

    @pl.when(cid == 1)
    def _():
        sweep(h_b)

    pltpu.sync_copy(mxl.at[pl.ds(0, G)], mx_o.at[cid, sid])
    pltpu.sync_copy(sml.at[pl.ds(0, G)], sm_o.at[cid, sid])
    pltpu.sync_copy(ctl.at[pl.ds(0, G)], ct_o.at[cid, sid])


def _blk(i):
    return (i, 0)


def _const(i):
    return (0, 0)


_ROW_SPEC = pl.BlockSpec((TCB, LANE), _blk)
_MAT_SPEC = pl.BlockSpec((LANE, LANE), _const)
_ROWV_SHAPE = jax.ShapeDtypeStruct((NP8, LANE), jnp.float32)


def _prep_body(cnt_ref, x_ref, bda_ref, bdb_ref, dinv_ref, ya_ref, yb_ref):
    deg = cnt_ref[0] + cnt_ref[1] + 1.0
    dinv = lax.rsqrt(deg)
    dinv_ref[...] = dinv
    ya_ref[...] = jnp.dot(x_ref[...], bda_ref[...],
                          preferred_element_type=jnp.float32) * dinv
    yb_ref[...] = jnp.dot(x_ref[...], bdb_ref[...],
                          preferred_element_type=jnp.float32) * dinv


_prep_tc = pl.pallas_call(
    _prep_body,
    grid=(NP8 // TCB,),
    in_specs=[
        pl.BlockSpec((NC, TCB, LANE), lambda i: (0, i, 0)),
        _ROW_SPEC, _MAT_SPEC, _MAT_SPEC,
    ],
    out_specs=[_ROW_SPEC, _ROW_SPEC, _ROW_SPEC],
    out_shape=[_ROWV_SHAPE, _ROWV_SHAPE, _ROWV_SHAPE],
)


def _mid1_body(aa_ref, ab_ref, ya_ref, yb_ref, dinv_ref, ba_ref, bb_ref,
               w_aa, w_ab, w_ba, w_bb,
               ha_ref, hb_ref, y2a_ref, y2b_ref):
    dinv = dinv_ref[...]
    ha = jnp.maximum((aa_ref[...] + ya_ref[...]) * dinv + ba_ref[...], 0.0)
    hb = jnp.maximum((ab_ref[...] + yb_ref[...]) * dinv + bb_ref[...], 0.0)
    ha_ref[...] = ha
    hb_ref[...] = hb
    y2a_ref[...] = (jnp.dot(ha, w_aa[...], preferred_element_type=jnp.float32)
                    + jnp.dot(hb, w_ba[...],
                              preferred_element_type=jnp.float32)) * dinv
    y2b_ref[...] = (jnp.dot(ha, w_ab[...], preferred_element_type=jnp.float32)
                    + jnp.dot(hb, w_bb[...],
                              preferred_element_type=jnp.float32)) * dinv


_mid1_tc = pl.pallas_call(
    _mid1_body,
    grid=(NP8 // TCB,),
    in_specs=[
        _ROW_SPEC, _ROW_SPEC, _ROW_SPEC, _ROW_SPEC, _ROW_SPEC,
        pl.BlockSpec((1, LANE), _const), pl.BlockSpec((1, LANE), _const),
        _MAT_SPEC, _MAT_SPEC, _MAT_SPEC, _MAT_SPEC,
    ],
    out_specs=[_ROW_SPEC, _ROW_SPEC, _ROW_SPEC, _ROW_SPEC],
    out_shape=[_ROWV_SHAPE, _ROWV_SHAPE, _ROWV_SHAPE, _ROWV_SHAPE],
)


def _mid2_body(aa_ref, ab_ref, ya_ref, yb_ref, dinv_ref, ba_ref, bb_ref,
               ha_ref, hb_ref):
    dinv = dinv_ref[...]
    ha_ref[...] = jnp.maximum(
        (aa_ref[...] + ya_ref[...]) * dinv + ba_ref[...], 0.0)
    hb_ref[...] = jnp.maximum(
        (ab_ref[...] + yb_ref[...]) * dinv + bb_ref[...], 0.0)


_mid2_tc = pl.pallas_call(
    _mid2_body,
    grid=(NP8 // TCB,),
    in_specs=[
        _ROW_SPEC, _ROW_SPEC, _ROW_SPEC, _ROW_SPEC, _ROW_SPEC,
        pl.BlockSpec((1, LANE), _const), pl.BlockSpec((1, LANE), _const),
    ],
    out_specs=[_ROW_SPEC, _ROW_SPEC],
    out_shape=[_ROWV_SHAPE, _ROWV_SHAPE],
)


def _head_body(mx1_ref, sm1_ref, mx2_ref, sm2_ref, ct_ref,
               lw1_ref, lb1_ref, lw2_ref, lb2_ref, out_ref):
    mx1 = mx1_ref[...]
    sm1 = sm1_ref[...]
    mx2 = mx2_ref[...]
    sm2 = sm2_ref[...]
    cnt = jnp.maximum(jnp.sum(ct_ref[...], axis=0), 1.0)
    parts = [
        jnp.max(mx1[0], axis=0) + jnp.max(mx2[0], axis=0),
        jnp.max(mx1[1], axis=0) + jnp.max(mx2[1], axis=0),
        (jnp.sum(sm1[0], axis=0) + jnp.sum(sm2[0], axis=0)) / cnt,
        (jnp.sum(sm1[1], axis=0) + jnp.sum(sm2[1], axis=0)) / cnt,
    ]
    lw1 = lw1_ref[...]
    z = lb1_ref[...]
    for p, xp in enumerate(parts):
        z = z + jnp.dot(xp, lw1[16 * p:16 * p + 16, :],
                        preferred_element_type=jnp.float32)
    z = jnp.maximum(z, 0.0)
    out_ref[...] = jnp.dot(z, lw2_ref[...],
                           preferred_element_type=jnp.float32) + lb2_ref[...]


_PART_SPEC = pl.BlockSpec((NC, NS, G, 16), lambda: (0, 0, 0, 0))

_head_tc = pl.pallas_call(
    _head_body,
    in_specs=[
        _PART_SPEC, _PART_SPEC, _PART_SPEC, _PART_SPEC,
        pl.BlockSpec((NS, G, 16), lambda: (0, 0, 0)),
        pl.BlockSpec((64, 64), lambda: (0, 0)),
        pl.BlockSpec((1, 64), lambda: (0, 0)),
        pl.BlockSpec((64, 64), lambda: (0, 0)),
        pl.BlockSpec((1, 64), lambda: (0, 0)),
    ],
    out_specs=pl.BlockSpec((G, CFG), lambda: (0, 0)),
    out_shape=jax.ShapeDtypeStruct((G, CFG), jnp.float32),
)


def _bd(w16):
    return jnp.kron(jnp.eye(8, dtype=jnp.float32), w16)


def kernel(x, edge_index, batch, W1, b1, W2, b2, LW1, Lb1, LW2, Lb2):
    src = edge_index[0]
    dst = edge_index[1]
    pad = N + (jnp.arange(E_PAD - E, dtype=jnp.int32) % 256)
    src_m = jnp.concatenate([src, pad])
    dst_m = jnp.concatenate([dst, pad])
    batch_p = jnp.concatenate(
        [batch, jnp.full((N_PAD - N,), G, jnp.int32)])

    x16 = jnp.zeros((N_PAD, 16), jnp.float32).at[:N, :3].set(x)
    x_pk = x16.reshape(NP8, LANE)

    W1p = jnp.zeros((16, 32), jnp.float32).at[:3].set(W1)
    bd1a = _bd(W1p[:, :16])
    bd1b = _bd(W1p[:, 16:])
    bd2aa = _bd(W2[:16, :16])
    bd2ab = _bd(W2[:16, 16:])
    bd2ba = _bd(W2[16:, :16])
    bd2bb = _bd(W2[16:, 16:])
    b1a = jnp.tile(b1[:16], 8).reshape(1, LANE)
    b1b = jnp.tile(b1[16:], 8).reshape(1, LANE)
    b2a = jnp.tile(b2[:16], 8).reshape(1, LANE)
    b2b = jnp.tile(b2[16:], 8).reshape(1, LANE)

    cnt = _deg_kernel(dst_m)
    cnt_pk = jnp.broadcast_to(cnt[:, :, None], (NC, N_PAD, 16)).reshape(
        NC, NP8, LANE)

    dinv_pk, y1a_pk, y1b_pk = _prep_tc(cnt_pk, x_pk, bd1a, bd1b)

    a1a, a1b = _agg_kernel(
        y1a_pk.reshape(N_PAD, 16), y1b_pk.reshape(N_PAD, 16), src_m, dst_m)

    h1a_pk, h1b_pk, y2a_pk, y2b_pk = _mid1_tc(
        a1a.reshape(NP8, LANE), a1b.reshape(NP8, LANE), y1a_pk, y1b_pk,
        dinv_pk, b1a, b1b, bd2aa, bd2ab, bd2ba, bd2bb)

    mx1, sm1, ct1 = _pool_kernel(
        h1a_pk.reshape(N_PAD, 16), h1b_pk.reshape(N_PAD, 16), batch_p)

    a2a, a2b = _agg_kernel(
        y2a_pk.reshape(N_PAD, 16), y2b_pk.reshape(N_PAD, 16), src_m, dst_m)

    h2a_pk, h2b_pk = _mid2_tc(
        a2a.reshape(NP8, LANE), a2b.reshape(NP8, LANE), y2a_pk, y2b_pk,
        dinv_pk, b2a, b2b)

    mx2, sm2, ct2 = _pool_kernel(
        h2a_pk.reshape(N_PAD, 16), h2b_pk.reshape(N_PAD, 16), batch_p)

    return _head_tc(mx1, sm1, mx2, sm2, ct1[0],
                    LW1, Lb1.reshape(1, 64), LW2, Lb2.reshape(1, 64))


# speedup vs baseline: 84.2052x; 1.8417x over previous
import functools

import jax
import jax.numpy as jnp
from jax import lax
from jax.experimental import pallas as pl
from jax.experimental.pallas import tpu as pltpu
from jax.experimental.pallas import tpu_sc as plsc

N = 100000
E = 3200000
G = 128
CFG = 64

NC = 2
NS = 16
LANE = 128

N_PAD = 100352
NP8 = N_PAD // 8
E_ROWS = 25344
E_PAD = E_ROWS * LANE

AGG_CH = 4
AGG_ROWS_PT = E_ROWS // NS
AGG_NCH = AGG_ROWS_PT // AGG_CH
AGG_CL = AGG_CH * LANE

DEG_CH = 12
DEG_ROWS_PT = E_ROWS // (NC * NS)
DEG_NCH = DEG_ROWS_PT // DEG_CH

ACC_ROWS_PT = N_PAD // NS
ZCH = 392

RCH = 784
GP = 132

TCB = 256
NEG = -3.4e38

_mesh = plsc.VectorSubcoreMesh(
    core_axis_name="c", subcore_axis_name="s", num_cores=NC, num_subcores=NS)

_sc_params = pltpu.CompilerParams(use_tc_tiling_on_sc=False)


def _zero_f32(ref, nrows):
    def body(r, _):
        ref[r] = jnp.zeros((16,), jnp.float32)
        return 0
    lax.fori_loop(0, nrows, body, 0)


@functools.partial(
    pl.kernel,
    out_type=jax.ShapeDtypeStruct((NC, N_PAD), jnp.float32),
    mesh=_mesh,
    compiler_params=_sc_params,
    scratch_types=[
        pltpu.VMEM_SHARED((N_PAD,), jnp.float32),
        pltpu.VMEM((DEG_CH * LANE,), jnp.int32),
        pltpu.VMEM((DEG_CH * LANE,), jnp.int32),
        pltpu.VMEM((DEG_CH * LANE,), jnp.float32),
        pltpu.VMEM((ACC_ROWS_PT,), jnp.float32),
        pltpu.SemaphoreType.DMA,
        pltpu.SemaphoreType.DMA,
    ],
)
def _deg_kernel(dst_v, out, acc, di0, di1, ones, zb, sem0, sem1):
    cid = lax.axis_index("c")
    sid = lax.axis_index("s")
    w = cid * NS + sid

    def obody(i, _):
        ones[pl.ds(i * 16, 16)] = jnp.ones((16,), jnp.float32)
        return 0
    lax.fori_loop(0, DEG_CH * LANE // 16, obody, 0)

    def zbody(i, _):
        zb[pl.ds(i * 16, 16)] = jnp.zeros((16,), jnp.float32)
        return 0
    lax.fori_loop(0, ACC_ROWS_PT // 16, zbody, 0)
    pltpu.sync_copy(zb, acc.at[pl.ds(sid * ACC_ROWS_PT, ACC_ROWS_PT)])
    plsc.subcore_barrier()

    def stage(c, di):
        base = (w * DEG_ROWS_PT + c * DEG_CH) * LANE
        pltpu.sync_copy(dst_v.at[pl.ds(base, DEG_CH * LANE)], di)

    def fire(di, sem):
        pltpu.async_copy(ones, acc.at[di], sem, add=True)

    def drain(di, sem):
        pltpu.make_async_copy(ones, acc.at[di], sem).wait()

    def pair(k, _):
        c = 2 * k

        @pl.when(k > 0)
        def _():
            drain(di0, sem0)
        stage(c, di0)
        fire(di0, sem0)

        @pl.when(k > 0)
        def _():
            drain(di1, sem1)
        stage(c + 1, di1)
        fire(di1, sem1)
        return 0
    lax.fori_loop(0, DEG_NCH // 2, pair, 0)
    drain(di0, sem0)
    drain(di1, sem1)

    plsc.subcore_barrier()
    pltpu.sync_copy(acc.at[pl.ds(sid * ACC_ROWS_PT, ACC_ROWS_PT)], zb)
    pltpu.sync_copy(zb, out.at[cid, pl.ds(sid * ACC_ROWS_PT, ACC_ROWS_PT)])


@functools.partial(
    pl.kernel,
    out_type=(jax.ShapeDtypeStruct((N_PAD, 16), jnp.float32),
              jax.ShapeDtypeStruct((N_PAD, 16), jnp.float32)),
    mesh=_mesh,
    compiler_params=_sc_params,
    scratch_types=[
        pltpu.VMEM_SHARED((N_PAD, 16), jnp.float32),
        pltpu.VMEM((AGG_CL,), jnp.int32),
        pltpu.VMEM((AGG_CL,), jnp.int32),
        pltpu.VMEM((AGG_CL,), jnp.int32),
        pltpu.VMEM((AGG_CL,), jnp.int32),
        pltpu.VMEM((AGG_CL,), jnp.int32),
        pltpu.VMEM((AGG_CL,), jnp.int32),
        pltpu.VMEM((AGG_CL,), jnp.int32),
        pltpu.VMEM((AGG_CL,), jnp.int32),
        pltpu.VMEM((AGG_CL, 16), jnp.float32),
        pltpu.VMEM((AGG_CL, 16), jnp.float32),
        pltpu.VMEM((AGG_CL, 16), jnp.float32),
        pltpu.SemaphoreType.DMA,
        pltpu.SemaphoreType.DMA,
        pltpu.SemaphoreType.DMA,
        pltpu.SemaphoreType.DMA,
        pltpu.SemaphoreType.DMA,
        pltpu.SemaphoreType.DMA,
        pltpu.SemaphoreType.DMA,
        pltpu.SemaphoreType.DMA,
        pltpu.SemaphoreType.DMA,
        pltpu.SemaphoreType.DMA,
    ],
)
def _agg_kernel(y_a, y_b, src_v, dst_v, out_a, out_b,
                acc, si0, si1, si2, si3, di0, di1, di2, di3,
                ro0, ro1, ro2,
                gs0, gs1, gs2, ss0, ss1, ss2, is0, is1, is2, is3):
    cid = lax.axis_index("c")
    sid = lax.axis_index("s")
    SIS = (si0, si1, si2, si3)
    DIS = (di0, di1, di2, di3)
    ROS = (ro0, ro1, ro2)
    GS = (gs0, gs1, gs2)
    SS = (ss0, ss1, ss2)
    IS = (is0, is1, is2, is3)

    _zero_f32(ro0, ZCH)
    for k in range(ACC_ROWS_PT // ZCH):
        pltpu.sync_copy(
            ro0.at[pl.ds(0, ZCH)],
            acc.at[pl.ds(sid * ACC_ROWS_PT + k * ZCH, ZCH)])
    plsc.subcore_barrier()

    def gather_scatter(y_ref):
        def fire_idx(c, e):
            base = (sid * AGG_ROWS_PT + c * AGG_CH) * LANE
            pltpu.async_copy(src_v.at[pl.ds(base, AGG_CL)], SIS[e], IS[e])
            pltpu.async_copy(dst_v.at[pl.ds(base, AGG_CL)], DIS[e], IS[e])

        def wait_idx(c, e):
            base = (sid * AGG_ROWS_PT + c * AGG_CH) * LANE
            pltpu.make_async_copy(
                src_v.at[pl.ds(base, AGG_CL)], SIS[e], IS[e]).wait()
            pltpu.make_async_copy(
                dst_v.at[pl.ds(base, AGG_CL)], DIS[e], IS[e]).wait()

        def fire_g(e, p):
            pltpu.async_copy(y_ref.at[SIS[e]], ROS[p], GS[p])

        def wait_g(e, p):
            pltpu.make_async_copy(y_ref.at[SIS[e]], ROS[p], GS[p]).wait()

        def fire_s(e, p):
            pltpu.async_copy(ROS[p], acc.at[DIS[e]], SS[p], add=True)

        def wait_s(e, p):
            pltpu.make_async_copy(ROS[p], acc.at[DIS[e]], SS[p]).wait()

        fire_idx(0, 0)

        def outer(k, _):
            c0 = 12 * k
            for j in range(12):
                c = c0 + j
                p = j % 3
                e = j % 4
                e3 = (j - 3) % 4
                e2 = (j - 2) % 4
                p2 = (j - 2) % 3
                if j >= 3:
                    wait_s(e3, p)
                else:
                    @pl.when(k > 0)
                    def _(e3=e3, p=p):
                        wait_s(e3, p)
                wait_idx(c, e)
                fire_g(e, p)
                if j == 11:
                    @pl.when(k < AGG_NCH // 12 - 1)
                    def _():
                        fire_idx(c + 1, (j + 1) % 4)
                else:
                    fire_idx(c + 1, (j + 1) % 4)
                if j >= 2:
                    wait_g(e2, p2)
                    fire_s(e2, p2)
                else:
                    @pl.when(k > 0)
                    def _(e2=e2, p2=p2):
                        wait_g(e2, p2)
                        fire_s(e2, p2)
            return 0
        lax.fori_loop(0, AGG_NCH // 12, outer, 0)

        n = AGG_NCH
        wait_g((n - 2) % 4, (n - 2) % 3)
        fire_s((n - 2) % 4, (n - 2) % 3)
        wait_g((n - 1) % 4, (n - 1) % 3)
        fire_s((n - 1) % 4, (n - 1) % 3)
        wait_s((n - 3) % 4, (n - 3) % 3)
        wait_s((n - 2) % 4, (n - 2) % 3)
        wait_s((n - 1) % 4, (n - 1) % 3)

    @pl.when(cid == 0)
    def _():
        gather_scatter(y_a)

    @pl.when(cid == 1)
    def _():
        gather_scatter(y_b)

    plsc.subcore_barrier()

    def writeback(out_ref):
        for k in range(ACC_ROWS_PT // ZCH):
            row0 = sid * ACC_ROWS_PT + k * ZCH
            pltpu.sync_copy(acc.at[pl.ds(row0, ZCH)], ro0.at[pl.ds(0, ZCH)])
            pltpu.sync_copy(ro0.at[pl.ds(0, ZCH)],
                            out_ref.at[pl.ds(row0, ZCH)])

    @pl.when(cid == 0)
    def _():
        writeback(out_a)

    @pl.when(cid == 1)
    def _():
        writeback(out_b)


_POOL_OUT_T = (
    jax.ShapeDtypeStruct((NC, NS, G, 16), jnp.float32),
    jax.ShapeDtypeStruct((NC, NS, G, 16), jnp.float32),
    jax.ShapeDtypeStruct((NC, NS, G, 16), jnp.float32),
)


@functools.partial(
    pl.kernel,
    out_type=_POOL_OUT_T,
    mesh=_mesh,
    compiler_params=_sc_params,
    scratch_types=[
        pltpu.VMEM((RCH, 16), jnp.float32),
        pltpu.VMEM((RCH,), jnp.int32),
        pltpu.VMEM((GP, 16), jnp.float32),
        pltpu.VMEM((GP, 16), jnp.float32),
        pltpu.VMEM((GP, 16), jnp.float32),
    ],
)
def _pool_kernel(h_a, h_b, batch_p, mx_o, sm_o, ct_o,
                 hbuf, bbuf, mxl, sml, ctl):
    cid = lax.axis_index("c")
    sid = lax.axis_index("s")

    def init(r, _):
        mxl[r] = jnp.full((16,), NEG, jnp.float32)
        sml[r] = jnp.zeros((16,), jnp.float32)
        ctl[r] = jnp.zeros((16,), jnp.float32)
        return 0
    lax.fori_loop(0, GP, init, 0)

    def sweep(h_ref):
        def chunk_body(k, _):
            row0 = sid * ACC_ROWS_PT + k * RCH
            pltpu.sync_copy(h_ref.at[pl.ds(row0, RCH)], hbuf)
            pltpu.sync_copy(batch_p.at[pl.ds(row0, RCH)], bbuf)

            def group_body(j, _):
                base = j * 16
                gv = bbuf[pl.ds(base, 16)]
                g0 = gv[0]
                g15 = gv[15]

                @pl.when(g0 == g15)
                def _():
                    v = [hbuf[base + i] for i in range(16)]
                    m = v
                    while len(m) > 1:
                        m = [jnp.maximum(m[2 * t], m[2 * t + 1])
                             for t in range(len(m) // 2)]
                    s = v
                    while len(s) > 1:
                        s = [s[2 * t] + s[2 * t + 1]
                             for t in range(len(s) // 2)]
                    mxl[g0] = jnp.maximum(mxl[g0], m[0])
                    sml[g0] = sml[g0] + s[0]
                    ctl[g0] = ctl[g0] + jnp.full((16,), 16.0, jnp.float32)

                @pl.when(g0 != g15)
                def _():
                    for i in range(16):
                        g = gv[i]
                        h = hbuf[base + i]
                        mxl[g] = jnp.maximum(mxl[g], h)
                        sml[g] = sml[g] + h
                        ctl[g] = ctl[g] + jnp.ones((16,), jnp.float32)
                return 0
            lax.fori_loop(0, RCH // 16, group_body, 0)
            return 0
        lax.fori_loop(0, ACC_ROWS_PT // RCH, chunk_body, 0)

    @pl.when(cid == 0)
    def _():
        sweep(h_a)

    @pl.when(cid == 1)
    def _():
        sweep(h_b)

    pltpu.sync_copy(mxl.at[pl.ds(0, G)], mx_o.at[cid, sid])
    pltpu.sync_copy(sml.at[pl.ds(0, G)], sm_o.at[cid, sid])
    pltpu.sync_copy(ctl.at[pl.ds(0, G)], ct_o.at[cid, sid])


def _blk(i):
    return (i, 0)


def _const(i):
    return (0, 0)


_ROW_SPEC = pl.BlockSpec((TCB, LANE), _blk)
_MAT_SPEC = pl.BlockSpec((LANE, LANE), _const)
_ROWV_SHAPE = jax.ShapeDtypeStruct((NP8, LANE), jnp.float32)


def _prep_body(cnt_ref, x_ref, bda_ref, bdb_ref, dinv_ref, ya_ref, yb_ref):
    deg = cnt_ref[0] + cnt_ref[1] + 1.0
    dinv = lax.rsqrt(deg)
    dinv_ref[...] = dinv
    ya_ref[...] = jnp.dot(x_ref[...], bda_ref[...],
                          preferred_element_type=jnp.float32) * dinv
    yb_ref[...] = jnp.dot(x_ref[...], bdb_ref[...],
                          preferred_element_type=jnp.float32) * dinv


_prep_tc = pl.pallas_call(
    _prep_body,
    grid=(NP8 // TCB,),
    in_specs=[
        pl.BlockSpec((NC, TCB, LANE), lambda i: (0, i, 0)),
        _ROW_SPEC, _MAT_SPEC, _MAT_SPEC,
    ],
    out_specs=[_ROW_SPEC, _ROW_SPEC, _ROW_SPEC],
    out_shape=[_ROWV_SHAPE, _ROWV_SHAPE, _ROWV_SHAPE],
)


def _mid1_body(aa_ref, ab_ref, ya_ref, yb_ref, dinv_ref, ba_ref, bb_ref,
               w_aa, w_ab, w_ba, w_bb,
               ha_ref, hb_ref, y2a_ref, y2b_ref):
    dinv = dinv_ref[...]
    ha = jnp.maximum((aa_ref[...] + ya_ref[...]) * dinv + ba_ref[...], 0.0)
    hb = jnp.maximum((ab_ref[...] + yb_ref[...]) * dinv + bb_ref[...], 0.0)
    ha_ref[...] = ha
    hb_ref[...] = hb
    y2a_ref[...] = (jnp.dot(ha, w_aa[...], preferred_element_type=jnp.float32)
                    + jnp.dot(hb, w_ba[...],
                              preferred_element_type=jnp.float32)) * dinv
    y2b_ref[...] = (jnp.dot(ha, w_ab[...], preferred_element_type=jnp.float32)
                    + jnp.dot(hb, w_bb[...],
                              preferred_element_type=jnp.float32)) * dinv


_mid1_tc = pl.pallas_call(
    _mid1_body,
    grid=(NP8 // TCB,),
    in_specs=[
        _ROW_SPEC, _ROW_SPEC, _ROW_SPEC, _ROW_SPEC, _ROW_SPEC,
        pl.BlockSpec((1, LANE), _const), pl.BlockSpec((1, LANE), _const),
        _MAT_SPEC, _MAT_SPEC, _MAT_SPEC, _MAT_SPEC,
    ],
    out_specs=[_ROW_SPEC, _ROW_SPEC, _ROW_SPEC, _ROW_SPEC],
    out_shape=[_ROWV_SHAPE, _ROWV_SHAPE, _ROWV_SHAPE, _ROWV_SHAPE],
)


def _mid2_body(aa_ref, ab_ref, ya_ref, yb_ref, dinv_ref, ba_ref, bb_ref,
               ha_ref, hb_ref):
    dinv = dinv_ref[...]
    ha_ref[...] = jnp.maximum(
        (aa_ref[...] + ya_ref[...]) * dinv + ba_ref[...], 0.0)
    hb_ref[...] = jnp.maximum(
        (ab_ref[...] + yb_ref[...]) * dinv + bb_ref[...], 0.0)


_mid2_tc = pl.pallas_call(
    _mid2_body,
    grid=(NP8 // TCB,),
    in_specs=[
        _ROW_SPEC, _ROW_SPEC, _ROW_SPEC, _ROW_SPEC, _ROW_SPEC,
        pl.BlockSpec((1, LANE), _const), pl.BlockSpec((1, LANE), _const),
    ],
    out_specs=[_ROW_SPEC, _ROW_SPEC],
    out_shape=[_ROWV_SHAPE, _ROWV_SHAPE],
)


def _head_body(mx1_ref, sm1_ref, mx2_ref, sm2_ref, ct_ref,
               lw1_ref, lb1_ref, lw2_ref, lb2_ref, out_ref):
    mx1 = mx1_ref[...]
    sm1 = sm1_ref[...]
    mx2 = mx2_ref[...]
    sm2 = sm2_ref[...]
    cnt = jnp.maximum(jnp.sum(ct_ref[...], axis=0), 1.0)
    parts = [
        jnp.max(mx1[0], axis=0) + jnp.max(mx2[0], axis=0),
        jnp.max(mx1[1], axis=0) + jnp.max(mx2[1], axis=0),
        (jnp.sum(sm1[0], axis=0) + jnp.sum(sm2[0], axis=0)) / cnt,
        (jnp.sum(sm1[1], axis=0) + jnp.sum(sm2[1], axis=0)) / cnt,
    ]
    lw1 = lw1_ref[...]
    z = lb1_ref[...]
    for p, xp in enumerate(parts):
        z = z + jnp.dot(xp, lw1[16 * p:16 * p + 16, :],
                        preferred_element_type=jnp.float32)
    z = jnp.maximum(z, 0.0)
    out_ref[...] = jnp.dot(z, lw2_ref[...],
                           preferred_element_type=jnp.float32) + lb2_ref[...]


_PART_SPEC = pl.BlockSpec((NC, NS, G, 16), lambda: (0, 0, 0, 0))

_head_tc = pl.pallas_call(
    _head_body,
    in_specs=[
        _PART_SPEC, _PART_SPEC, _PART_SPEC, _PART_SPEC,
        pl.BlockSpec((NS, G, 16), lambda: (0, 0, 0)),
        pl.BlockSpec((64, 64), lambda: (0, 0)),
        pl.BlockSpec((1, 64), lambda: (0, 0)),
        pl.BlockSpec((64, 64), lambda: (0, 0)),
        pl.BlockSpec((1, 64), lambda: (0, 0)),
    ],
    out_specs=pl.BlockSpec((G, CFG), lambda: (0, 0)),
    out_shape=jax.ShapeDtypeStruct((G, CFG), jnp.float32),
)


def _bd(w16):
    return jnp.kron(jnp.eye(8, dtype=jnp.float32), w16)


def kernel(x, edge_index, batch, W1, b1, W2, b2, LW1, Lb1, LW2, Lb2):
    src = edge_index[0]
    dst = edge_index[1]
    pad = N + (jnp.arange(E_PAD - E, dtype=jnp.int32) % 256)
    src_m = jnp.concatenate([src, pad])
    dst_m = jnp.concatenate([dst, pad])
    batch_p = jnp.concatenate(
        [batch, jnp.full((N_PAD - N,), G, jnp.int32)])

    x16 = jnp.zeros((N_PAD, 16), jnp.float32).at[:N, :3].set(x)
    x_pk = x16.reshape(NP8, LANE)

    W1p = jnp.zeros((16, 32), jnp.float32).at[:3].set(W1)
    bd1a = _bd(W1p[:, :16])
    bd1b = _bd(W1p[:, 16:])
    bd2aa = _bd(W2[:16, :16])
    bd2ab = _bd(W2[:16, 16:])
    bd2ba = _bd(W2[16:, :16])
    bd2bb = _bd(W2[16:, 16:])
    b1a = jnp.tile(b1[:16], 8).reshape(1, LANE)
    b1b = jnp.tile(b1[16:], 8).reshape(1, LANE)
    b2a = jnp.tile(b2[:16], 8).reshape(1, LANE)
    b2b = jnp.tile(b2[16:], 8).reshape(1, LANE)

    cnt = _deg_kernel(dst_m)
    cnt_pk = jnp.broadcast_to(cnt[:, :, None], (NC, N_PAD, 16)).reshape(
        NC, NP8, LANE)

    dinv_pk, y1a_pk, y1b_pk = _prep_tc(cnt_pk, x_pk, bd1a, bd1b)

    a1a, a1b = _agg_kernel(
        y1a_pk.reshape(N_PAD, 16), y1b_pk.reshape(N_PAD, 16), src_m, dst_m)

    h1a_pk, h1b_pk, y2a_pk, y2b_pk = _mid1_tc(
        a1a.reshape(NP8, LANE), a1b.reshape(NP8, LANE), y1a_pk, y1b_pk,
        dinv_pk, b1a, b1b, bd2aa, bd2ab, bd2ba, bd2bb)

    mx1, sm1, ct1 = _pool_kernel(
        h1a_pk.reshape(N_PAD, 16), h1b_pk.reshape(N_PAD, 16), batch_p)

    a2a, a2b = _agg_kernel(
        y2a_pk.reshape(N_PAD, 16), y2b_pk.reshape(N_PAD, 16), src_m, dst_m)

    h2a_pk, h2b_pk = _mid2_tc(
        a2a.reshape(NP8, LANE), a2b.reshape(NP8, LANE), y2a_pk, y2b_pk,
        dinv_pk, b2a, b2b)

    mx2, sm2, ct2 = _pool_kernel(
        h2a_pk.reshape(N_PAD, 16), h2b_pk.reshape(N_PAD, 16), batch_p)

    return _head_tc(mx1, sm1, mx2, sm2, ct1[0],
                    LW1, Lb1.reshape(1, 64), LW2, Lb2.reshape(1, 64))
